# baseline (speedup 1.0000x reference)
import functools

import jax
import jax.numpy as jnp
from jax import lax
from jax.experimental import pallas as pl
from jax.experimental.pallas import tpu as pltpu
from jax.experimental.pallas import tpu_sc as plsc

N_NODES = 10000
E = 320000
D = 128
NC, NS = 2, 16
CHUNK = 80
N_CHUNKS = E // CHUNK
CHUNKS_PER_TEC = N_CHUNKS // NS
ROWS_PER_TEC = N_NODES // NS


def _sc_segment_sum(table, src_idx, dst_idx, zeros_acc, zeros_cnt, ones_blk):
  mesh = plsc.VectorSubcoreMesh(core_axis_name="c", subcore_axis_name="s",
                                num_cores=NC, num_subcores=NS)

  @functools.partial(
      pl.kernel,
      out_type=(
          jax.ShapeDtypeStruct((2 * N_NODES, D), jnp.float32),
          jax.ShapeDtypeStruct((2 * N_NODES, 16), jnp.float32),
      ),
      mesh=mesh,
      scratch_types=[
          [pltpu.VMEM((CHUNK,), jnp.int32)] * 4,
          [pltpu.VMEM((CHUNK,), jnp.int32)] * 4,
          [pltpu.VMEM((CHUNK, D), jnp.float32)] * 3,
          pltpu.VMEM((CHUNK, 16), jnp.float32),
          pltpu.VMEM_SHARED((N_NODES, D), jnp.float32),
          pltpu.VMEM_SHARED((N_NODES, 16), jnp.float32),
          [pltpu.SemaphoreType.DMA] * 4,
          [pltpu.SemaphoreType.DMA] * 3,
          [pltpu.SemaphoreType.DMA] * 3,
      ],
      compiler_params=pltpu.CompilerParams(use_tc_tiling_on_sc=False),
  )
  def k(table_h, srci_h, dsti_h, zacc_h, zcnt_h, ones_h,
        acc_out_h, cnt_out_h,
        srci, dsti, rows, ones_v,
        acc_sh, cnt_sh, sem_i, sem_g, sem_s):
    c = lax.axis_index("c")
    s = lax.axis_index("s")
    r0 = s * ROWS_PER_TEC

    pltpu.sync_copy(zacc_h.at[pl.ds(r0, ROWS_PER_TEC)],
                    acc_sh.at[pl.ds(r0, ROWS_PER_TEC)])
    pltpu.sync_copy(zcnt_h.at[pl.ds(r0, ROWS_PER_TEC)],
                    cnt_sh.at[pl.ds(r0, ROWS_PER_TEC)])
    pltpu.sync_copy(ones_h, ones_v)
    plsc.subcore_barrier()

    ebase = c * E

    def _off(kk):
      return ebase + (kk * NS + s) * CHUNK

    def _valid(kk):
      return jnp.logical_and(kk >= 0, kk * NS + s < N_CHUNKS)

    def idx_start(kk, i):
      @pl.when(_valid(kk))
      def _():
        off = _off(kk)
        pltpu.async_copy(srci_h.at[pl.ds(off, CHUNK)], srci[i], sem_i[i])
        pltpu.async_copy(dsti_h.at[pl.ds(off, CHUNK)], dsti[i], sem_i[i])

    def idx_wait(kk, i):
      @pl.when(_valid(kk))
      def _():
        off = _off(kk)
        pltpu.make_async_copy(srci_h.at[pl.ds(off, CHUNK)], srci[i],
                              sem_i[i]).wait()
        pltpu.make_async_copy(dsti_h.at[pl.ds(off, CHUNK)], dsti[i],
                              sem_i[i]).wait()

    def gather_start(kk, r, i):
      @pl.when(_valid(kk))
      def _():
        pltpu.async_copy(table_h.at[srci[i]], rows[r], sem_g[r])

    def gather_wait(kk, r, i):
      @pl.when(_valid(kk))
      def _():
        pltpu.make_async_copy(table_h.at[srci[i]], rows[r], sem_g[r]).wait()

    def scatter_start(kk, r, i):
      @pl.when(_valid(kk))
      def _():
        pltpu.async_copy(rows[r], acc_sh.at[dsti[i]], sem_s[r], add=True)
        pltpu.async_copy(ones_v, cnt_sh.at[dsti[i]], sem_s[r], add=True)

    def scatter_wait(kk, r, i):
      @pl.when(_valid(kk))
      def _():
        pltpu.make_async_copy(rows[r], acc_sh.at[dsti[i]], sem_s[r]).wait()
        pltpu.make_async_copy(ones_v, cnt_sh.at[dsti[i]], sem_s[r]).wait()

    idx_start(0, 0)
    idx_start(1, 1)
    idx_start(2, 2)
    idx_wait(0, 0)
    gather_start(0, 0, 0)
    idx_wait(1, 1)
    gather_start(1, 1, 1)

    def group_body(g, carry):
      for i in range(12):
        kk = g * 12 + i
        gather_wait(kk, i % 3, i % 4)
        scatter_start(kk, i % 3, i % 4)
        scatter_wait(kk - 1, (i - 1) % 3, (i - 1) % 4)
        idx_start(kk + 3, (i + 3) % 4)
        idx_wait(kk + 2, (i + 2) % 4)
        gather_start(kk + 2, (i + 2) % 3, (i + 2) % 4)
      return carry

    n_groups = -(-(CHUNKS_PER_TEC + 1) // 12)
    lax.fori_loop(0, n_groups, group_body, 0)
    plsc.subcore_barrier()

    out0 = c * N_NODES + r0
    pltpu.sync_copy(acc_sh.at[pl.ds(r0, ROWS_PER_TEC)],
                    acc_out_h.at[pl.ds(out0, ROWS_PER_TEC)])
    pltpu.sync_copy(cnt_sh.at[pl.ds(r0, ROWS_PER_TEC)],
                    cnt_out_h.at[pl.ds(out0, ROWS_PER_TEC)])

  return k(table, src_idx, dst_idx, zeros_acc, zeros_cnt, ones_blk)


_BR = 2000


def _tc_body(acc_r, cnt_r, xd_r, wlt_r, bl_r, wrt_r, wv_r, bb_r, out_r):
  cnt = jnp.sum(cnt_r[...], axis=1) * (1.0 / 16.0)
  mean = acc_r[...] / jnp.maximum(cnt, 1.0)[:, None]
  h = (jnp.dot(mean, wlt_r[0], preferred_element_type=jnp.float32)
       + bl_r[0]
       + jnp.dot(xd_r[...], wrt_r[0], preferred_element_type=jnp.float32))
  h = jnp.maximum(h, 0.0)
  z = jnp.dot(h, wv_r[0], preferred_element_type=jnp.float32) + bb_r[0]
  out_r[...] = jnp.where(z >= 0, z, 0.001 * z)[None, :, :]


def _tc_dense(acc, cnt, x_dst, WlT, bl, WrT, wv, bb):
  nb = N_NODES // _BR
  return pl.pallas_call(
      _tc_body,
      grid=(2, nb),
      in_specs=[
          pl.BlockSpec((_BR, D), lambda r, i: (r * nb + i, 0)),
          pl.BlockSpec((_BR, 16), lambda r, i: (r * nb + i, 0)),
          pl.BlockSpec((_BR, D), lambda r, i: (r * nb + i, 0)),
          pl.BlockSpec((1, D, D), lambda r, i: (r, 0, 0)),
          pl.BlockSpec((1, 1, D), lambda r, i: (r, 0, 0)),
          pl.BlockSpec((1, D, D), lambda r, i: (r, 0, 0)),
          pl.BlockSpec((1, D, 1), lambda r, i: (r, 0, 0)),
          pl.BlockSpec((1, 1, 1), lambda r, i: (r, 0, 0)),
      ],
      out_specs=pl.BlockSpec((1, _BR, 1), lambda r, i: (r, i, 0)),
      out_shape=jax.ShapeDtypeStruct((2, N_NODES, 1), jnp.float32),
  )(acc, cnt, x_dst, WlT, bl, WrT, wv, bb)


def kernel(x_pfas_sites, x_gw_wells, edge_index_sites_to_wells,
           edge_index_wells_to_sites, Wl_s2w, bl_s2w, Wr_s2w,
           Wl_w2s, bl_w2s, Wr_w2s, W_gw, b_gw, W_sites, b_sites):
  e1 = edge_index_sites_to_wells.astype(jnp.int32)
  e2 = edge_index_wells_to_sites.astype(jnp.int32)
  table = jnp.concatenate([x_pfas_sites, x_gw_wells], axis=0)
  src_idx = jnp.concatenate([e1[0], e2[0] + N_NODES])
  dst_idx = jnp.concatenate([e1[1], e2[1]])

  zeros_acc = jnp.zeros((N_NODES, D), jnp.float32)
  zeros_cnt = jnp.zeros((N_NODES, 16), jnp.float32)
  ones_blk = jnp.ones((CHUNK, 16), jnp.float32)

  acc, cnt = _sc_segment_sum(table, src_idx, dst_idx,
                             zeros_acc, zeros_cnt, ones_blk)

  x_dst = jnp.concatenate([x_gw_wells, x_pfas_sites], axis=0)
  WlT = jnp.stack([Wl_s2w.T, Wl_w2s.T])
  WrT = jnp.stack([Wr_s2w.T, Wr_w2s.T])
  bl = jnp.stack([bl_s2w, bl_w2s])[:, None, :]
  wv = jnp.stack([W_gw[0], W_sites[0]])[:, :, None]
  bb = jnp.stack([b_gw, b_sites])[:, :, None]

  out = _tc_dense(acc, cnt, x_dst, WlT, bl, WrT, wv, bb)
  return (out[0], out[1])


# speedup vs baseline: 1.0272x; 1.0272x over previous
import functools

import jax
import jax.numpy as jnp
from jax import lax
from jax.experimental import pallas as pl
from jax.experimental.pallas import tpu as pltpu
from jax.experimental.pallas import tpu_sc as plsc

N_NODES = 10000
E = 320000
D = 128
NC, NS = 2, 16
CHUNK = 80
N_CHUNKS = E // CHUNK
CHUNKS_PER_TEC = N_CHUNKS // NS
ROWS_PER_TEC = N_NODES // NS


def _sc_segment_sum(table, src_idx, dst_idx, zeros_acc, zeros_cnt, ones_blk):
  mesh = plsc.VectorSubcoreMesh(core_axis_name="c", subcore_axis_name="s",
                                num_cores=NC, num_subcores=NS)

  @functools.partial(
      pl.kernel,
      out_type=(
          jax.ShapeDtypeStruct((2 * N_NODES, D), jnp.float32),
          jax.ShapeDtypeStruct((2 * N_NODES, 16), jnp.float32),
      ),
      mesh=mesh,
      scratch_types=[
          [pltpu.VMEM((CHUNK,), jnp.int32)] * 4,
          [pltpu.VMEM((CHUNK,), jnp.int32)] * 4,
          [pltpu.VMEM((CHUNK, D), jnp.float32)] * 3,
          pltpu.VMEM((CHUNK, 16), jnp.float32),
          pltpu.VMEM_SHARED((N_NODES, D), jnp.float32),
          pltpu.VMEM_SHARED((N_NODES, 16), jnp.float32),
          [pltpu.SemaphoreType.DMA] * 4,
          [pltpu.SemaphoreType.DMA] * 3,
          [pltpu.SemaphoreType.DMA] * 3,
      ],
      compiler_params=pltpu.CompilerParams(use_tc_tiling_on_sc=False),
  )
  def k(table_h, srci_h, dsti_h, zacc_h, zcnt_h, ones_h,
        acc_out_h, cnt_out_h,
        srci, dsti, rows, ones_v,
        acc_sh, cnt_sh, sem_i, sem_g, sem_s):
    c = lax.axis_index("c")
    s = lax.axis_index("s")
    r0 = s * ROWS_PER_TEC

    pltpu.sync_copy(zacc_h.at[pl.ds(r0, ROWS_PER_TEC)],
                    acc_sh.at[pl.ds(r0, ROWS_PER_TEC)])
    pltpu.sync_copy(zcnt_h.at[pl.ds(r0, ROWS_PER_TEC)],
                    cnt_sh.at[pl.ds(r0, ROWS_PER_TEC)])
    pltpu.sync_copy(ones_h, ones_v)
    plsc.subcore_barrier()

    ebase = c * E

    def _off(kk):
      return ebase + (kk * NS + s) * CHUNK

    def _valid(kk):
      return jnp.logical_and(kk >= 0, kk * NS + s < N_CHUNKS)

    def idx_start(kk, i):
      @pl.when(_valid(kk))
      def _():
        off = _off(kk)
        pltpu.async_copy(srci_h.at[pl.ds(off, CHUNK)], srci[i], sem_i[i])
        pltpu.async_copy(dsti_h.at[pl.ds(off, CHUNK)], dsti[i], sem_i[i])

    def idx_wait(kk, i):
      @pl.when(_valid(kk))
      def _():
        off = _off(kk)
        pltpu.make_async_copy(srci_h.at[pl.ds(off, CHUNK)], srci[i],
                              sem_i[i]).wait()
        pltpu.make_async_copy(dsti_h.at[pl.ds(off, CHUNK)], dsti[i],
                              sem_i[i]).wait()

    def gather_start(kk, r, i):
      @pl.when(_valid(kk))
      def _():
        pltpu.async_copy(table_h.at[srci[i]], rows[r], sem_g[r])

    def gather_wait(kk, r, i):
      @pl.when(_valid(kk))
      def _():
        pltpu.make_async_copy(table_h.at[srci[i]], rows[r], sem_g[r]).wait()

    def scatter_start(kk, r, i):
      @pl.when(_valid(kk))
      def _():
        pltpu.async_copy(rows[r], acc_sh.at[dsti[i]], sem_s[r], add=True)
        pltpu.async_copy(ones_v, cnt_sh.at[dsti[i]], sem_s[r], add=True)

    def scatter_wait(kk, r, i):
      @pl.when(_valid(kk))
      def _():
        pltpu.make_async_copy(rows[r], acc_sh.at[dsti[i]], sem_s[r]).wait()
        pltpu.make_async_copy(ones_v, cnt_sh.at[dsti[i]], sem_s[r]).wait()

    idx_start(0, 0)
    idx_start(1, 1)
    idx_start(2, 2)
    idx_wait(0, 0)
    gather_start(0, 0, 0)
    idx_wait(1, 1)
    gather_start(1, 1, 1)

    def group_body(g, carry):
      for i in range(12):
        kk = g * 12 + i
        gather_wait(kk, i % 3, i % 4)
        scatter_start(kk, i % 3, i % 4)
        scatter_wait(kk - 1, (i - 1) % 3, (i - 1) % 4)
        idx_start(kk + 3, (i + 3) % 4)
        idx_wait(kk + 2, (i + 2) % 4)
        gather_start(kk + 2, (i + 2) % 3, (i + 2) % 4)
      return carry

    n_groups = -(-(CHUNKS_PER_TEC + 1) // 12)
    lax.fori_loop(0, n_groups, group_body, 0)
    plsc.subcore_barrier()

    out0 = c * N_NODES + r0
    pltpu.sync_copy(acc_sh.at[pl.ds(r0, ROWS_PER_TEC)],
                    acc_out_h.at[pl.ds(out0, ROWS_PER_TEC)])
    pltpu.sync_copy(cnt_sh.at[pl.ds(r0, ROWS_PER_TEC)],
                    cnt_out_h.at[pl.ds(out0, ROWS_PER_TEC)])

  return k(table, src_idx, dst_idx, zeros_acc, zeros_cnt, ones_blk)


_BR = 2000


def _tc_body(acc_r, cnt_r, xd_r, wlt_r, bl_r, wrt_r, wv_r, bb_r, out_r):
  cnt = jnp.sum(cnt_r[...], axis=1) * (1.0 / 16.0)
  mean = acc_r[...] / jnp.maximum(cnt, 1.0)[:, None]
  h = (jnp.dot(mean, wlt_r[0], preferred_element_type=jnp.float32)
       + bl_r[0]
       + jnp.dot(xd_r[...], wrt_r[0], preferred_element_type=jnp.float32))
  h = jnp.maximum(h, 0.0)
  z = jnp.dot(h, wv_r[0], preferred_element_type=jnp.float32) + bb_r[0]
  out_r[...] = jnp.where(z >= 0, z, 0.001 * z)[None, :, :]


def _tc_dense(acc, cnt, x_dst, WlT, bl, WrT, wv, bb):
  nb = N_NODES // _BR
  return pl.pallas_call(
      _tc_body,
      grid=(2, nb),
      in_specs=[
          pl.BlockSpec((_BR, D), lambda r, i: (r * nb + i, 0)),
          pl.BlockSpec((_BR, 16), lambda r, i: (r * nb + i, 0)),
          pl.BlockSpec((_BR, D), lambda r, i: ((1 - r) * nb + i, 0)),
          pl.BlockSpec((1, D, D), lambda r, i: (r, 0, 0)),
          pl.BlockSpec((1, 1, D), lambda r, i: (r, 0, 0)),
          pl.BlockSpec((1, D, D), lambda r, i: (r, 0, 0)),
          pl.BlockSpec((1, D, 1), lambda r, i: (r, 0, 0)),
          pl.BlockSpec((1, 1, 1), lambda r, i: (r, 0, 0)),
      ],
      out_specs=pl.BlockSpec((1, _BR, 1), lambda r, i: (r, i, 0)),
      out_shape=jax.ShapeDtypeStruct((2, N_NODES, 1), jnp.float32),
  )(acc, cnt, x_dst, WlT, bl, WrT, wv, bb)


def kernel(x_pfas_sites, x_gw_wells, edge_index_sites_to_wells,
           edge_index_wells_to_sites, Wl_s2w, bl_s2w, Wr_s2w,
           Wl_w2s, bl_w2s, Wr_w2s, W_gw, b_gw, W_sites, b_sites):
  e1 = edge_index_sites_to_wells.astype(jnp.int32)
  e2 = edge_index_wells_to_sites.astype(jnp.int32)
  table = jnp.concatenate([x_pfas_sites, x_gw_wells], axis=0)
  src_idx = jnp.concatenate([e1[0], e2[0] + N_NODES])
  dst_idx = jnp.concatenate([e1[1], e2[1]])

  zeros_acc = jnp.zeros((N_NODES, D), jnp.float32)
  zeros_cnt = jnp.zeros((N_NODES, 16), jnp.float32)
  ones_blk = jnp.ones((CHUNK, 16), jnp.float32)

  acc, cnt = _sc_segment_sum(table, src_idx, dst_idx,
                             zeros_acc, zeros_cnt, ones_blk)

  x_dst = table
  WlT = jnp.stack([Wl_s2w.T, Wl_w2s.T])
  WrT = jnp.stack([Wr_s2w.T, Wr_w2s.T])
  bl = jnp.stack([bl_s2w, bl_w2s])[:, None, :]
  wv = jnp.stack([W_gw[0], W_sites[0]])[:, :, None]
  bb = jnp.stack([b_gw, b_sites])[:, :, None]

  out = _tc_dense(acc, cnt, x_dst, WlT, bl, WrT, wv, bb)
  return (out[0], out[1])


# speedup vs baseline: 1.0448x; 1.0171x over previous
import functools

import jax
import jax.numpy as jnp
from jax import lax
from jax.experimental import pallas as pl
from jax.experimental.pallas import tpu as pltpu
from jax.experimental.pallas import tpu_sc as plsc

N_NODES = 10000
E = 320000
D = 128
NC, NS = 2, 16
CHUNK = 80
N_CHUNKS = E // CHUNK
CHUNKS_PER_TEC = N_CHUNKS // NS
ROWS_PER_TEC = N_NODES // NS


def _sc_segment_sum(table, src_idx, dst_idx, ones_blk):
  mesh = plsc.VectorSubcoreMesh(core_axis_name="c", subcore_axis_name="s",
                                num_cores=NC, num_subcores=NS)

  @functools.partial(
      pl.kernel,
      out_type=(
          jax.ShapeDtypeStruct((2 * N_NODES, D), jnp.float32),
          jax.ShapeDtypeStruct((2 * N_NODES, 16), jnp.float32),
      ),
      mesh=mesh,
      scratch_types=[
          [pltpu.VMEM((CHUNK,), jnp.int32)] * 4,
          [pltpu.VMEM((CHUNK,), jnp.int32)] * 4,
          [pltpu.VMEM((CHUNK, D), jnp.float32)] * 3,
          pltpu.VMEM((CHUNK, 16), jnp.float32),
          pltpu.VMEM_SHARED((N_NODES, D), jnp.float32),
          pltpu.VMEM_SHARED((N_NODES, 16), jnp.float32),
          [pltpu.SemaphoreType.DMA] * 4,
          [pltpu.SemaphoreType.DMA] * 3,
          [pltpu.SemaphoreType.DMA] * 3,
      ],
      compiler_params=pltpu.CompilerParams(use_tc_tiling_on_sc=False),
  )
  def k(table_h, srci_h, dsti_h, ones_h,
        acc_out_h, cnt_out_h,
        srci, dsti, rows, ones_v,
        acc_sh, cnt_sh, sem_i, sem_g, sem_s):
    c = lax.axis_index("c")
    s = lax.axis_index("s")
    r0 = s * ROWS_PER_TEC

    zv = jnp.zeros((16,), jnp.float32)

    def zrow(i2, carry):
      for j in range(8):
        rows[0][i2, pl.ds(j * 16, 16)] = zv
      return carry

    lax.fori_loop(0, CHUNK, zrow, 0)
    nfull = ROWS_PER_TEC // CHUNK
    rem = ROWS_PER_TEC - nfull * CHUNK
    for b in range(nfull):
      pltpu.sync_copy(rows[0], acc_sh.at[pl.ds(r0 + b * CHUNK, CHUNK)])
      pltpu.sync_copy(rows[0].at[:, pl.ds(0, 16)],
                      cnt_sh.at[pl.ds(r0 + b * CHUNK, CHUNK)])
    pltpu.sync_copy(rows[0].at[pl.ds(0, rem)],
                    acc_sh.at[pl.ds(r0 + nfull * CHUNK, rem)])
    pltpu.sync_copy(rows[0].at[pl.ds(0, rem), pl.ds(0, 16)],
                    cnt_sh.at[pl.ds(r0 + nfull * CHUNK, rem)])
    pltpu.sync_copy(ones_h, ones_v)
    plsc.subcore_barrier()

    ebase = c * E

    def _off(kk):
      return ebase + (kk * NS + s) * CHUNK

    def _valid(kk):
      return jnp.logical_and(kk >= 0, kk * NS + s < N_CHUNKS)

    def idx_start(kk, i):
      @pl.when(_valid(kk))
      def _():
        off = _off(kk)
        pltpu.async_copy(srci_h.at[pl.ds(off, CHUNK)], srci[i], sem_i[i])
        pltpu.async_copy(dsti_h.at[pl.ds(off, CHUNK)], dsti[i], sem_i[i])

    def idx_wait(kk, i):
      @pl.when(_valid(kk))
      def _():
        off = _off(kk)
        pltpu.make_async_copy(srci_h.at[pl.ds(off, CHUNK)], srci[i],
                              sem_i[i]).wait()
        pltpu.make_async_copy(dsti_h.at[pl.ds(off, CHUNK)], dsti[i],
                              sem_i[i]).wait()

    def gather_start(kk, r, i):
      @pl.when(_valid(kk))
      def _():
        pltpu.async_copy(table_h.at[srci[i]], rows[r], sem_g[r])

    def gather_wait(kk, r, i):
      @pl.when(_valid(kk))
      def _():
        pltpu.make_async_copy(table_h.at[srci[i]], rows[r], sem_g[r]).wait()

    def scatter_start(kk, r, i):
      @pl.when(_valid(kk))
      def _():
        pltpu.async_copy(rows[r], acc_sh.at[dsti[i]], sem_s[r], add=True)
        pltpu.async_copy(ones_v, cnt_sh.at[dsti[i]], sem_s[r], add=True)

    def scatter_wait(kk, r, i):
      @pl.when(_valid(kk))
      def _():
        pltpu.make_async_copy(rows[r], acc_sh.at[dsti[i]], sem_s[r]).wait()
        pltpu.make_async_copy(ones_v, cnt_sh.at[dsti[i]], sem_s[r]).wait()

    idx_start(0, 0)
    idx_start(1, 1)
    idx_start(2, 2)
    idx_wait(0, 0)
    gather_start(0, 0, 0)
    idx_wait(1, 1)
    gather_start(1, 1, 1)

    def group_body(g, carry):
      for i in range(12):
        kk = g * 12 + i
        gather_wait(kk, i % 3, i % 4)
        scatter_start(kk, i % 3, i % 4)
        scatter_wait(kk - 1, (i - 1) % 3, (i - 1) % 4)
        idx_start(kk + 3, (i + 3) % 4)
        idx_wait(kk + 2, (i + 2) % 4)
        gather_start(kk + 2, (i + 2) % 3, (i + 2) % 4)
      return carry

    n_groups = -(-(CHUNKS_PER_TEC + 1) // 12)
    lax.fori_loop(0, n_groups, group_body, 0)
    plsc.subcore_barrier()

    out0 = c * N_NODES + r0
    pltpu.sync_copy(acc_sh.at[pl.ds(r0, ROWS_PER_TEC)],
                    acc_out_h.at[pl.ds(out0, ROWS_PER_TEC)])
    pltpu.sync_copy(cnt_sh.at[pl.ds(r0, ROWS_PER_TEC)],
                    cnt_out_h.at[pl.ds(out0, ROWS_PER_TEC)])

  return k(table, src_idx, dst_idx, ones_blk)


_BR = 2000


def _tc_body(acc_r, cnt_r, xd_r, wlt_r, bl_r, wrt_r, wv_r, bb_r, out_r):
  cnt = jnp.sum(cnt_r[...], axis=1) * (1.0 / 16.0)
  mean = acc_r[...] / jnp.maximum(cnt, 1.0)[:, None]
  h = (jnp.dot(mean, wlt_r[0], preferred_element_type=jnp.float32)
       + bl_r[0]
       + jnp.dot(xd_r[...], wrt_r[0], preferred_element_type=jnp.float32))
  h = jnp.maximum(h, 0.0)
  z = jnp.dot(h, wv_r[0], preferred_element_type=jnp.float32) + bb_r[0]
  out_r[...] = jnp.where(z >= 0, z, 0.001 * z)[None, :, :]


def _tc_dense(acc, cnt, x_dst, WlT, bl, WrT, wv, bb):
  nb = N_NODES // _BR
  return pl.pallas_call(
      _tc_body,
      grid=(2, nb),
      in_specs=[
          pl.BlockSpec((_BR, D), lambda r, i: (r * nb + i, 0)),
          pl.BlockSpec((_BR, 16), lambda r, i: (r * nb + i, 0)),
          pl.BlockSpec((_BR, D), lambda r, i: ((1 - r) * nb + i, 0)),
          pl.BlockSpec((1, D, D), lambda r, i: (r, 0, 0)),
          pl.BlockSpec((1, 1, D), lambda r, i: (r, 0, 0)),
          pl.BlockSpec((1, D, D), lambda r, i: (r, 0, 0)),
          pl.BlockSpec((1, D, 1), lambda r, i: (r, 0, 0)),
          pl.BlockSpec((1, 1, 1), lambda r, i: (r, 0, 0)),
      ],
      out_specs=pl.BlockSpec((1, _BR, 1), lambda r, i: (r, i, 0)),
      out_shape=jax.ShapeDtypeStruct((2, N_NODES, 1), jnp.float32),
  )(acc, cnt, x_dst, WlT, bl, WrT, wv, bb)


def kernel(x_pfas_sites, x_gw_wells, edge_index_sites_to_wells,
           edge_index_wells_to_sites, Wl_s2w, bl_s2w, Wr_s2w,
           Wl_w2s, bl_w2s, Wr_w2s, W_gw, b_gw, W_sites, b_sites):
  e1 = edge_index_sites_to_wells.astype(jnp.int32)
  e2 = edge_index_wells_to_sites.astype(jnp.int32)
  table = jnp.concatenate([x_pfas_sites, x_gw_wells], axis=0)
  src_idx = jnp.concatenate([e1[0], e2[0] + N_NODES])
  dst_idx = jnp.concatenate([e1[1], e2[1]])

  ones_blk = jnp.ones((CHUNK, 16), jnp.float32)

  acc, cnt = _sc_segment_sum(table, src_idx, dst_idx, ones_blk)

  x_dst = table
  WlT = jnp.stack([Wl_s2w.T, Wl_w2s.T])
  WrT = jnp.stack([Wr_s2w.T, Wr_w2s.T])
  bl = jnp.stack([bl_s2w, bl_w2s])[:, None, :]
  wv = jnp.stack([W_gw[0], W_sites[0]])[:, :, None]
  bb = jnp.stack([b_gw, b_sites])[:, :, None]

  out = _tc_dense(acc, cnt, x_dst, WlT, bl, WrT, wv, bb)
  return (out[0], out[1])
